# baseline (speedup 1.0000x reference)
import functools

import jax
import jax.numpy as jnp
from jax import lax
from jax.experimental import pallas as pl
from jax.experimental.pallas import tpu as pltpu
from jax.experimental.pallas import tpu_sc as plsc

N = 10000
E = 320000
D_IN = 128
D_HID = 256
D_OUT = 128

NC = 2
NS = 16
NW = NC * NS
C = 80
K = 128
NB = 4
GD = 2
E_PAD = NW * K * C
ROWS_PER_SUB = 632
N_PAD = NS * ROWS_PER_SUB


def _segsum_partial_sc(table, src_w, dst_w, zeros):
    mesh = plsc.VectorSubcoreMesh(core_axis_name="c", subcore_axis_name="s")

    @functools.partial(
        pl.kernel,
        out_type=jax.ShapeDtypeStruct((2 * N_PAD, D_IN), jnp.float32),
        mesh=mesh,
        scratch_types=[
            [pltpu.VMEM((1, C), jnp.int32) for _ in range(NB)],
            [pltpu.VMEM((1, C), jnp.int32) for _ in range(NB)],
            [pltpu.VMEM((C, D_IN), jnp.float32) for _ in range(NB)],
            pltpu.VMEM_SHARED((N_PAD, D_IN), jnp.float32),
            [pltpu.SemaphoreType.DMA for _ in range(NB)],
            [pltpu.SemaphoreType.DMA for _ in range(NB)],
            [pltpu.SemaphoreType.DMA for _ in range(NB)],
            [pltpu.SemaphoreType.DMA for _ in range(NB)],
        ],
    )
    def seg_kernel(table_hbm, src_hbm, dst_hbm, zeros_hbm, out_hbm,
                   srcv, dstv, rows_v, acc_sh, s_sem, d_sem, gsem, ssem):
        c = lax.axis_index("c")
        s = lax.axis_index("s")
        wid = s * NC + c

        row0 = s * ROWS_PER_SUB
        pltpu.sync_copy(zeros_hbm.at[pl.ds(row0, ROWS_PER_SUB)],
                        acc_sh.at[pl.ds(row0, ROWS_PER_SUB)])
        plsc.subcore_barrier()

        for v in range(NB):
            pltpu.async_copy(src_hbm.at[wid, pl.ds(v, 1)], srcv[v], s_sem[v])
        for v in range(GD):
            pltpu.async_copy(dst_hbm.at[wid, pl.ds(v, 1)], dstv[v], d_sem[v])
        for v in range(GD):
            pltpu.make_async_copy(src_hbm.at[wid, pl.ds(v, 1)],
                                  srcv[v], s_sem[v]).wait()
            pltpu.async_copy(table_hbm.at[srcv[v].at[0]], rows_v[v], gsem[v])

        def chunk_steps(i, w, u2, first):
            pltpu.make_async_copy(table_hbm.at[srcv[w].at[0]],
                                  rows_v[w], gsem[w]).wait()
            pltpu.async_copy(src_hbm.at[wid, pl.ds(i + NB, 1)],
                             srcv[w], s_sem[w])
            pltpu.make_async_copy(dst_hbm.at[wid, pl.ds(i, 1)],
                                  dstv[w], d_sem[w]).wait()
            pltpu.async_copy(rows_v[w], acc_sh.at[dstv[w].at[0]],
                             ssem[w], add=True)
            if not first:
                pltpu.make_async_copy(rows_v[u2],
                                      acc_sh.at[dstv[u2].at[0]],
                                      ssem[u2]).wait()
            pltpu.async_copy(dst_hbm.at[wid, pl.ds(i + GD, 1)],
                             dstv[u2], d_sem[u2])
            pltpu.make_async_copy(src_hbm.at[wid, pl.ds(i + GD, 1)],
                                  srcv[u2], s_sem[u2]).wait()
            pltpu.async_copy(table_hbm.at[srcv[u2].at[0]],
                             rows_v[u2], gsem[u2])

        for u in range(NB):
            chunk_steps(u, u, (u + GD) % NB, first=(u < NB - GD))

        def body(j, carry):
            for u in range(NB):
                i = NB * j + u
                chunk_steps(i, u, (u + GD) % NB, first=False)
            return carry

        lax.fori_loop(1, K // NB, body, 0, unroll=False)

        for v in range(GD):
            pltpu.make_async_copy(table_hbm.at[srcv[v].at[0]],
                                  rows_v[v], gsem[v]).wait()
        for v in range(GD, NB):
            pltpu.make_async_copy(rows_v[v], acc_sh.at[dstv[v].at[0]],
                                  ssem[v]).wait()
        for v in range(GD, NB):
            pltpu.make_async_copy(src_hbm.at[wid, pl.ds(K + v, 1)],
                                  srcv[v], s_sem[v]).wait()
        for v in range(GD):
            pltpu.make_async_copy(dst_hbm.at[wid, pl.ds(K + v, 1)],
                                  dstv[v], d_sem[v]).wait()

        plsc.subcore_barrier()
        pltpu.sync_copy(acc_sh.at[pl.ds(row0, ROWS_PER_SUB)],
                        out_hbm.at[pl.ds(c * N_PAD + row0, ROWS_PER_SUB)])

    return seg_kernel(table, src_w, dst_w, zeros)


def _layer1_tc(p0, p1, x, w1_rel, b1, w1_root, w2_rel):
    BLK = 2000

    def body(p0_ref, p1_ref, x_ref, w1rel_ref, b1_ref, w1root_ref, w2rel_ref,
             h_ref, g2_ref):
        agg = p0_ref[...] + p1_ref[...]
        acc = jnp.dot(agg, w1rel_ref[...], preferred_element_type=jnp.float32)
        acc += jnp.dot(x_ref[...], w1root_ref[...],
                       preferred_element_type=jnp.float32)
        h = jnp.maximum(acc + b1_ref[...], 0.0)
        h_ref[...] = h
        g2_ref[...] = jnp.dot(h, w2rel_ref[...],
                              preferred_element_type=jnp.float32)

    grid = N // BLK
    row_blk = lambda i: (i, 0)
    rep = lambda i: (0, 0)
    return pl.pallas_call(
        body,
        grid=(grid,),
        in_specs=[
            pl.BlockSpec((BLK, D_IN), row_blk),
            pl.BlockSpec((BLK, D_IN), row_blk),
            pl.BlockSpec((BLK, D_IN), row_blk),
            pl.BlockSpec((D_IN, D_HID), rep),
            pl.BlockSpec((1, D_HID), lambda i: (0, 0)),
            pl.BlockSpec((D_IN, D_HID), rep),
            pl.BlockSpec((D_HID, D_OUT), rep),
        ],
        out_specs=[
            pl.BlockSpec((BLK, D_HID), row_blk),
            pl.BlockSpec((BLK, D_OUT), row_blk),
        ],
        out_shape=[
            jax.ShapeDtypeStruct((N, D_HID), jnp.float32),
            jax.ShapeDtypeStruct((N, D_OUT), jnp.float32),
        ],
    )(p0, p1, x, w1_rel, b1.reshape(1, D_HID), w1_root, w2_rel)


def _layer2_tc(p0, p1, h, b2, w2_root):
    BLK = 2000

    def body(p0_ref, p1_ref, h_ref, b2_ref, w2root_ref, out_ref):
        acc = jnp.dot(h_ref[...], w2root_ref[...],
                      preferred_element_type=jnp.float32)
        out_ref[...] = p0_ref[...] + p1_ref[...] + b2_ref[...] + acc

    grid = N // BLK
    row_blk = lambda i: (i, 0)
    return pl.pallas_call(
        body,
        grid=(grid,),
        in_specs=[
            pl.BlockSpec((BLK, D_OUT), row_blk),
            pl.BlockSpec((BLK, D_OUT), row_blk),
            pl.BlockSpec((BLK, D_HID), row_blk),
            pl.BlockSpec((1, D_OUT), lambda i: (0, 0)),
            pl.BlockSpec((D_HID, D_OUT), lambda i: (0, 0)),
        ],
        out_specs=pl.BlockSpec((BLK, D_OUT), row_blk),
        out_shape=jax.ShapeDtypeStruct((N, D_OUT), jnp.float32),
    )(p0, p1, h, b2.reshape(1, D_OUT), w2_root)


def kernel(x, edge_index, W1_rel, b1, W1_root, W2_rel, b2, W2_root):
    ei = edge_index.astype(jnp.int32)
    pad = E_PAD - E
    src = jnp.concatenate([ei[0], jnp.zeros((pad,), jnp.int32)])
    dst = jnp.concatenate([ei[1], jnp.full((pad,), N, jnp.int32)])
    src_w = jnp.concatenate(
        [src.reshape(NW, K, C), jnp.zeros((NW, NB, C), jnp.int32)], axis=1)
    dst_w = jnp.concatenate(
        [dst.reshape(NW, K, C), jnp.full((NW, NB, C), N, jnp.int32)], axis=1)
    zeros = jnp.zeros((N_PAD, D_IN), jnp.float32)

    p1 = _segsum_partial_sc(x, src_w, dst_w, zeros)
    h, g2 = _layer1_tc(p1[:N], p1[N_PAD:N_PAD + N], x,
                       W1_rel, b1, W1_root, W2_rel)
    p2 = _segsum_partial_sc(g2, src_w, dst_w, zeros)
    out = _layer2_tc(p2[:N], p2[N_PAD:N_PAD + N], h, b2, W2_root)
    return out


# speedup vs baseline: 1.8108x; 1.8108x over previous
import functools

import jax
import jax.numpy as jnp
from jax import lax
from jax.experimental import pallas as pl
from jax.experimental.pallas import tpu as pltpu
from jax.experimental.pallas import tpu_sc as plsc

N = 10000
E = 320000
D_IN = 128
D_HID = 256
D_OUT = 128

NC = 2
NS = 16
NW = NC * NS
C = 112
K = 90
NB = 2
GD = 1
E_PAD = NW * K * C
ROWS_PER_SUB = 632
N_PAD = NS * ROWS_PER_SUB


def _segsum_partial_sc(table, src_w, dst_w, zeros):
    mesh = plsc.VectorSubcoreMesh(core_axis_name="c", subcore_axis_name="s")

    @functools.partial(
        pl.kernel,
        out_type=jax.ShapeDtypeStruct((2 * N_PAD, D_IN), jnp.float32),
        mesh=mesh,
        scratch_types=[
            [pltpu.VMEM((1, C), jnp.int32) for _ in range(NB)],
            [pltpu.VMEM((1, C), jnp.int32) for _ in range(NB)],
            [pltpu.VMEM((C, D_IN), jnp.float32) for _ in range(NB)],
            pltpu.VMEM_SHARED((N_PAD, D_IN), jnp.float32),
            [pltpu.SemaphoreType.DMA for _ in range(NB)],
            [pltpu.SemaphoreType.DMA for _ in range(NB)],
            [pltpu.SemaphoreType.DMA for _ in range(NB)],
            [pltpu.SemaphoreType.DMA for _ in range(NB)],
        ],
    )
    def seg_kernel(table_hbm, src_hbm, dst_hbm, zeros_hbm, out_hbm,
                   srcv, dstv, rows_v, acc_sh, s_sem, d_sem, gsem, ssem):
        c = lax.axis_index("c")
        s = lax.axis_index("s")
        wid = s * NC + c

        row0 = s * ROWS_PER_SUB
        pltpu.sync_copy(zeros_hbm.at[pl.ds(row0, ROWS_PER_SUB)],
                        acc_sh.at[pl.ds(row0, ROWS_PER_SUB)])
        plsc.subcore_barrier()

        for v in range(NB):
            pltpu.async_copy(src_hbm.at[wid, pl.ds(v, 1)], srcv[v], s_sem[v])
        for v in range(GD):
            pltpu.async_copy(dst_hbm.at[wid, pl.ds(v, 1)], dstv[v], d_sem[v])
        for v in range(GD):
            pltpu.make_async_copy(src_hbm.at[wid, pl.ds(v, 1)],
                                  srcv[v], s_sem[v]).wait()
            pltpu.async_copy(table_hbm.at[srcv[v].at[0]], rows_v[v], gsem[v])

        def chunk_steps(i, w, u2, first):
            pltpu.make_async_copy(table_hbm.at[srcv[w].at[0]],
                                  rows_v[w], gsem[w]).wait()
            pltpu.async_copy(src_hbm.at[wid, pl.ds(i + NB, 1)],
                             srcv[w], s_sem[w])
            pltpu.make_async_copy(dst_hbm.at[wid, pl.ds(i, 1)],
                                  dstv[w], d_sem[w]).wait()
            pltpu.async_copy(rows_v[w], acc_sh.at[dstv[w].at[0]],
                             ssem[w], add=True)
            if not first:
                pltpu.make_async_copy(rows_v[u2],
                                      acc_sh.at[dstv[u2].at[0]],
                                      ssem[u2]).wait()
            pltpu.async_copy(dst_hbm.at[wid, pl.ds(i + GD, 1)],
                             dstv[u2], d_sem[u2])
            pltpu.make_async_copy(src_hbm.at[wid, pl.ds(i + GD, 1)],
                                  srcv[u2], s_sem[u2]).wait()
            pltpu.async_copy(table_hbm.at[srcv[u2].at[0]],
                             rows_v[u2], gsem[u2])

        for u in range(NB):
            chunk_steps(u, u, (u + GD) % NB, first=(u < NB - GD))

        def body(j, carry):
            for u in range(NB):
                i = NB * j + u
                chunk_steps(i, u, (u + GD) % NB, first=False)
            return carry

        lax.fori_loop(1, K // NB, body, 0, unroll=False)

        for v in range(GD):
            pltpu.make_async_copy(table_hbm.at[srcv[v].at[0]],
                                  rows_v[v], gsem[v]).wait()
        for v in range(GD, NB):
            pltpu.make_async_copy(rows_v[v], acc_sh.at[dstv[v].at[0]],
                                  ssem[v]).wait()
        for v in range(GD, NB):
            pltpu.make_async_copy(src_hbm.at[wid, pl.ds(K + v, 1)],
                                  srcv[v], s_sem[v]).wait()
        for v in range(GD):
            pltpu.make_async_copy(dst_hbm.at[wid, pl.ds(K + v, 1)],
                                  dstv[v], d_sem[v]).wait()

        plsc.subcore_barrier()
        pltpu.sync_copy(acc_sh.at[pl.ds(row0, ROWS_PER_SUB)],
                        out_hbm.at[pl.ds(c * N_PAD + row0, ROWS_PER_SUB)])

    return seg_kernel(table, src_w, dst_w, zeros)


def _layer1_tc(p0, p1, x, w1_rel, b1, w1_root, w2_rel):
    BLK = 2000

    def body(p0_ref, p1_ref, x_ref, w1rel_ref, b1_ref, w1root_ref, w2rel_ref,
             h_ref, g2_ref):
        agg = p0_ref[...] + p1_ref[...]
        acc = jnp.dot(agg, w1rel_ref[...], preferred_element_type=jnp.float32)
        acc += jnp.dot(x_ref[...], w1root_ref[...],
                       preferred_element_type=jnp.float32)
        h = jnp.maximum(acc + b1_ref[...], 0.0)
        h_ref[...] = h
        g2_ref[...] = jnp.dot(h, w2rel_ref[...],
                              preferred_element_type=jnp.float32)

    grid = N // BLK
    row_blk = lambda i: (i, 0)
    rep = lambda i: (0, 0)
    return pl.pallas_call(
        body,
        grid=(grid,),
        in_specs=[
            pl.BlockSpec((BLK, D_IN), row_blk),
            pl.BlockSpec((BLK, D_IN), row_blk),
            pl.BlockSpec((BLK, D_IN), row_blk),
            pl.BlockSpec((D_IN, D_HID), rep),
            pl.BlockSpec((1, D_HID), lambda i: (0, 0)),
            pl.BlockSpec((D_IN, D_HID), rep),
            pl.BlockSpec((D_HID, D_OUT), rep),
        ],
        out_specs=[
            pl.BlockSpec((BLK, D_HID), row_blk),
            pl.BlockSpec((BLK, D_OUT), row_blk),
        ],
        out_shape=[
            jax.ShapeDtypeStruct((N, D_HID), jnp.float32),
            jax.ShapeDtypeStruct((N, D_OUT), jnp.float32),
        ],
    )(p0, p1, x, w1_rel, b1.reshape(1, D_HID), w1_root, w2_rel)


def _layer2_tc(p0, p1, h, b2, w2_root):
    BLK = 2000

    def body(p0_ref, p1_ref, h_ref, b2_ref, w2root_ref, out_ref):
        acc = jnp.dot(h_ref[...], w2root_ref[...],
                      preferred_element_type=jnp.float32)
        out_ref[...] = p0_ref[...] + p1_ref[...] + b2_ref[...] + acc

    grid = N // BLK
    row_blk = lambda i: (i, 0)
    return pl.pallas_call(
        body,
        grid=(grid,),
        in_specs=[
            pl.BlockSpec((BLK, D_OUT), row_blk),
            pl.BlockSpec((BLK, D_OUT), row_blk),
            pl.BlockSpec((BLK, D_HID), row_blk),
            pl.BlockSpec((1, D_OUT), lambda i: (0, 0)),
            pl.BlockSpec((D_HID, D_OUT), lambda i: (0, 0)),
        ],
        out_specs=pl.BlockSpec((BLK, D_OUT), row_blk),
        out_shape=jax.ShapeDtypeStruct((N, D_OUT), jnp.float32),
    )(p0, p1, h, b2.reshape(1, D_OUT), w2_root)


def kernel(x, edge_index, W1_rel, b1, W1_root, W2_rel, b2, W2_root):
    ei = edge_index.astype(jnp.int32)
    pad = E_PAD - E
    src = jnp.concatenate([ei[0], jnp.zeros((pad,), jnp.int32)])
    dst = jnp.concatenate([ei[1], jnp.full((pad,), N, jnp.int32)])
    src_w = jnp.concatenate(
        [src.reshape(NW, K, C), jnp.zeros((NW, NB, C), jnp.int32)], axis=1)
    dst_w = jnp.concatenate(
        [dst.reshape(NW, K, C), jnp.full((NW, NB, C), N, jnp.int32)], axis=1)
    zeros = jnp.zeros((N_PAD, D_IN), jnp.float32)

    p1 = _segsum_partial_sc(x, src_w, dst_w, zeros)
    h, g2 = _layer1_tc(p1[:N], p1[N_PAD:N_PAD + N], x,
                       W1_rel, b1, W1_root, W2_rel)
    p2 = _segsum_partial_sc(g2, src_w, dst_w, zeros)
    out = _layer2_tc(p2[:N], p2[N_PAD:N_PAD + N], h, b2, W2_root)
    return out
